# Initial kernel scaffold; baseline (speedup 1.0000x reference)
import jax
import jax.numpy as jnp
from jax.experimental import pallas as pl


def kernel(funcname_1, funcname_2, table, W, U, b):
    raise NotImplementedError("write your pallas kernel here")


# speedup vs baseline: 4.2614x; 4.2614x over previous
import functools

import jax
import jax.numpy as jnp
from jax import lax
from jax.experimental import pallas as pl
from jax.experimental.pallas import tpu as pltpu
from jax.experimental.pallas import tpu_sc as plsc

VOCAB = 100000
EMBED = 128
SEQ = 200
HID = 256
BATCH = 1024
B2 = 2 * BATCH

_NC, _NS = 2, 16
NW = _NC * _NS
TOTAL_ROWS = B2 * SEQ
IDX_ROWS = TOTAL_ROWS // 128
ROWS_PER_W = IDX_ROWS // NW
CHUNK = 128


def _sc_gather(table, idx3d):
    mesh = plsc.VectorSubcoreMesh(core_axis_name="c", subcore_axis_name="s")

    @functools.partial(
        pl.kernel,
        mesh=mesh,
        out_type=jax.ShapeDtypeStruct((TOTAL_ROWS, EMBED), jnp.float32),
        scratch_types=[
            pltpu.VMEM((ROWS_PER_W, CHUNK), jnp.int32),
            pltpu.VMEM((CHUNK, EMBED), jnp.float32),
            pltpu.SemaphoreType.DMA,
        ],
    )
    def k(table_hbm, idx_hbm, out_hbm, idx_v, rows_v, sem):
        wid = lax.axis_index("s") * _NC + lax.axis_index("c")
        base = wid * ROWS_PER_W
        pltpu.sync_copy(idx_hbm.at[wid], idx_v)

        def body(j, carry):
            pltpu.async_copy(table_hbm.at[idx_v.at[j]], rows_v, sem).wait()
            pltpu.sync_copy(rows_v, out_hbm.at[pl.ds((base + j) * CHUNK, CHUNK)])
            return carry

        lax.fori_loop(0, ROWS_PER_W, body, 0)

    return k(table, idx3d)


def _lstm_tc(x, idxT, W, U, b):
    def body(x_ref, idx_ref, w_ref, u_ref, b_ref, c1_ref, c2_ref, sim_ref,
             h_s, c_s):
        t = pl.program_id(0)

        @pl.when(t == 0)
        def _init():
            h_s[...] = jnp.zeros((B2, HID), jnp.float32)
            c_s[...] = jnp.zeros((B2, HID), jnp.float32)

        x_t = x_ref[0]
        z = jnp.dot(x_t, w_ref[...], preferred_element_type=jnp.float32)
        z = z + jnp.dot(h_s[...], u_ref[...], preferred_element_type=jnp.float32)
        z = z + b_ref[...]
        i = jax.nn.sigmoid(z[:, 0 * HID:1 * HID])
        f = jax.nn.sigmoid(z[:, 1 * HID:2 * HID])
        g = jnp.tanh(z[:, 2 * HID:3 * HID])
        o = jax.nn.sigmoid(z[:, 3 * HID:4 * HID])
        c_new = f * c_s[...] + i * g
        h_new = o * jnp.tanh(c_new)
        m = idx_ref[0] != 0
        c_s[...] = jnp.where(m, c_new, c_s[...])
        h_s[...] = jnp.where(m, h_new, h_s[...])

        @pl.when(t == SEQ - 1)
        def _fin():
            cc = c_s[...]
            a = cc[:BATCH]
            bb = cc[BATCH:]
            na = jnp.sum(a * a, axis=1, keepdims=True)
            nb = jnp.sum(bb * bb, axis=1, keepdims=True)
            ab = jnp.sum(a * bb, axis=1, keepdims=True)
            inv = lax.rsqrt(jnp.maximum(na, 1e-12) * jnp.maximum(nb, 1e-12))
            c1_ref[...] = a
            c2_ref[...] = bb
            sim_ref[...] = ab * inv

    grid = (SEQ,)
    return pl.pallas_call(
        body,
        grid=grid,
        in_specs=[
            pl.BlockSpec((1, B2, EMBED), lambda t: (t, 0, 0)),
            pl.BlockSpec((1, B2, 1), lambda t: (t, 0, 0)),
            pl.BlockSpec((EMBED, 4 * HID), lambda t: (0, 0)),
            pl.BlockSpec((HID, 4 * HID), lambda t: (0, 0)),
            pl.BlockSpec((1, 4 * HID), lambda t: (0, 0)),
        ],
        out_specs=[
            pl.BlockSpec((BATCH, HID), lambda t: (0, 0)),
            pl.BlockSpec((BATCH, HID), lambda t: (0, 0)),
            pl.BlockSpec((BATCH, 1), lambda t: (0, 0)),
        ],
        out_shape=[
            jax.ShapeDtypeStruct((BATCH, HID), jnp.float32),
            jax.ShapeDtypeStruct((BATCH, HID), jnp.float32),
            jax.ShapeDtypeStruct((BATCH, 1), jnp.float32),
        ],
        scratch_shapes=[
            pltpu.VMEM((B2, HID), jnp.float32),
            pltpu.VMEM((B2, HID), jnp.float32),
        ],
    )(x, idxT, W, U, b)


def kernel(funcname_1, funcname_2, table, W, U, b):
    idx = jnp.concatenate([funcname_1, funcname_2], axis=0).astype(jnp.int32)
    idxT = idx.T
    x = _sc_gather(table, idxT.reshape(NW, ROWS_PER_W, 128))
    c1, c2, sim = _lstm_tc(
        x.reshape(SEQ, B2, EMBED),
        idxT.reshape(SEQ, B2, 1),
        W, U, b.reshape(1, 4 * HID),
    )
    return (c1, c2, sim.reshape(BATCH))


# speedup vs baseline: 4.8333x; 1.1342x over previous
import functools

import jax
import jax.numpy as jnp
from jax import lax
from jax.experimental import pallas as pl
from jax.experimental.pallas import tpu as pltpu
from jax.experimental.pallas import tpu_sc as plsc

VOCAB = 100000
EMBED = 128
SEQ = 200
HID = 256
BATCH = 1024
B2 = 2 * BATCH

_NC, _NS = 2, 16
NW = _NC * _NS
TOTAL_ROWS = B2 * SEQ
IDX_ROWS = TOTAL_ROWS // 128
ROWS_PER_W = IDX_ROWS // NW
CHUNK = 128


def _sc_gather(table, idx3d):
    mesh = plsc.VectorSubcoreMesh(core_axis_name="c", subcore_axis_name="s")

    @functools.partial(
        pl.kernel,
        mesh=mesh,
        out_type=jax.ShapeDtypeStruct((TOTAL_ROWS, EMBED), jnp.float32),
        scratch_types=[
            pltpu.VMEM((ROWS_PER_W, CHUNK), jnp.int32),
            pltpu.VMEM((2 * CHUNK, EMBED), jnp.float32),
            pltpu.VMEM((2 * CHUNK, EMBED), jnp.float32),
            pltpu.SemaphoreType.DMA,
            pltpu.SemaphoreType.DMA,
        ],
    )
    def k(table_hbm, idx_hbm, out_hbm, idx_v, rows_a, rows_b, sem_a, sem_b):
        wid = lax.axis_index("s") * _NC + lax.axis_index("c")
        base = wid * ROWS_PER_W
        pltpu.sync_copy(idx_hbm.at[wid], idx_v)

        def fill(buf, sem, j):
            pltpu.async_copy(table_hbm.at[idx_v.at[j]],
                             buf.at[pl.ds(0, CHUNK)], sem)
            pltpu.async_copy(table_hbm.at[idx_v.at[j + 1]],
                             buf.at[pl.ds(CHUNK, CHUNK)], sem)

        def drain_store(buf, sem, j):
            pltpu.make_async_copy(table_hbm.at[pl.ds(0, 2 * CHUNK)], buf,
                                  sem).wait()
            pltpu.sync_copy(buf,
                            out_hbm.at[pl.ds((base + j) * CHUNK, 2 * CHUNK)])

        fill(rows_a, sem_a, 0)
        fill(rows_b, sem_b, 2)

        def body(jj, carry):
            j = 4 * jj
            drain_store(rows_a, sem_a, j)

            @pl.when(j + 4 < ROWS_PER_W)
            def _pa():
                fill(rows_a, sem_a, j + 4)

            drain_store(rows_b, sem_b, j + 2)

            @pl.when(j + 6 < ROWS_PER_W)
            def _pb():
                fill(rows_b, sem_b, j + 6)

            return carry

        lax.fori_loop(0, ROWS_PER_W // 4, body, 0)

    return k(table, idx3d)


def _lstm_tc(x, idxT, W, U, b):
    scale = jnp.concatenate([
        jnp.full((HID,), 0.5, jnp.float32),
        jnp.full((HID,), 0.5, jnp.float32),
        jnp.ones((HID,), jnp.float32),
        jnp.full((HID,), 0.5, jnp.float32),
    ])
    wu = (jnp.concatenate([W, U], axis=0) * scale[None, :]).astype(jnp.bfloat16)
    bs = (b * scale).reshape(1, 4 * HID)

    RT = min(64, B2)
    NRT = B2 // RT
    NCT = HID // 128

    def body(x_ref, idx_ref, wu_ref, b_ref, c1_ref, c2_ref, sim_ref,
             xh_s, c_s, z_s):
        t = pl.program_id(0)

        @pl.when(t == 0)
        def _init():
            xh_s[:, EMBED:] = jnp.zeros((B2, HID), jnp.bfloat16)
            c_s[...] = jnp.zeros((B2, HID), jnp.float32)

        xh_s[:, :EMBED] = x_ref[0].astype(jnp.bfloat16)
        z_s[...] = jnp.dot(xh_s[...], wu_ref[...],
                           preferred_element_type=jnp.float32)

        def tile(bt):
            rs = pl.ds(bt * RT, RT)
            m = idx_ref[0, rs, :] != 0
            for kc in range(NCT):
                col = pl.ds(kc * 128, 128)
                zi = z_s[rs, pl.ds(0 * HID + kc * 128, 128)] + b_ref[0, pl.ds(0 * HID + kc * 128, 128)]
                zf = z_s[rs, pl.ds(1 * HID + kc * 128, 128)] + b_ref[0, pl.ds(1 * HID + kc * 128, 128)]
                zg = z_s[rs, pl.ds(2 * HID + kc * 128, 128)] + b_ref[0, pl.ds(2 * HID + kc * 128, 128)]
                zo = z_s[rs, pl.ds(3 * HID + kc * 128, 128)] + b_ref[0, pl.ds(3 * HID + kc * 128, 128)]
                i = 0.5 * jnp.tanh(zi) + 0.5
                f = 0.5 * jnp.tanh(zf) + 0.5
                g = jnp.tanh(zg)
                o = 0.5 * jnp.tanh(zo) + 0.5
                c_old = c_s[rs, col]
                c_new = f * c_old + i * g
                h_new = o * jnp.tanh(c_new)
                c_s[rs, col] = jnp.where(m, c_new, c_old)
                hcol = pl.ds(EMBED + kc * 128, 128)
                xh_s[rs, hcol] = jnp.where(m, h_new.astype(jnp.bfloat16),
                                           xh_s[rs, hcol])

        for bt in range(NRT):
            tile(bt)

        @pl.when(t == SEQ - 1)
        def _fin():
            cc = c_s[...]
            a = cc[:BATCH]
            bb = cc[BATCH:]
            na = jnp.sum(a * a, axis=1, keepdims=True)
            nb = jnp.sum(bb * bb, axis=1, keepdims=True)
            ab = jnp.sum(a * bb, axis=1, keepdims=True)
            inv = lax.rsqrt(jnp.maximum(na, 1e-12) * jnp.maximum(nb, 1e-12))
            c1_ref[...] = a
            c2_ref[...] = bb
            sim_ref[...] = ab * inv

    grid = (SEQ,)
    return pl.pallas_call(
        body,
        grid=grid,
        in_specs=[
            pl.BlockSpec((1, B2, EMBED), lambda t: (t, 0, 0)),
            pl.BlockSpec((1, B2, 1), lambda t: (t, 0, 0)),
            pl.BlockSpec((EMBED + HID, 4 * HID), lambda t: (0, 0)),
            pl.BlockSpec((1, 4 * HID), lambda t: (0, 0)),
        ],
        out_specs=[
            pl.BlockSpec((BATCH, HID), lambda t: (0, 0)),
            pl.BlockSpec((BATCH, HID), lambda t: (0, 0)),
            pl.BlockSpec((BATCH, 1), lambda t: (0, 0)),
        ],
        out_shape=[
            jax.ShapeDtypeStruct((BATCH, HID), jnp.float32),
            jax.ShapeDtypeStruct((BATCH, HID), jnp.float32),
            jax.ShapeDtypeStruct((BATCH, 1), jnp.float32),
        ],
        scratch_shapes=[
            pltpu.VMEM((B2, EMBED + HID), jnp.bfloat16),
            pltpu.VMEM((B2, HID), jnp.float32),
            pltpu.VMEM((B2, 4 * HID), jnp.float32),
        ],
    )(x, idxT, wu, bs)


def kernel(funcname_1, funcname_2, table, W, U, b):
    idx = jnp.concatenate([funcname_1, funcname_2], axis=0).astype(jnp.int32)
    idxT = idx.T
    x = _sc_gather(table, idxT.reshape(NW, ROWS_PER_W, 128))
    c1, c2, sim = _lstm_tc(
        x.reshape(SEQ, B2, EMBED),
        idxT.reshape(SEQ, B2, 1),
        W, U, b,
    )
    return (c1, c2, sim.reshape(BATCH))
